# baseline (speedup 1.0000x reference)
import functools
import math

import jax
import jax.numpy as jnp
from jax import lax
from jax.experimental import pallas as pl
from jax.experimental.pallas import tpu as pltpu
from jax.experimental.pallas import tpu_sc as plsc


def _sc_geometry():
    try:
        info = plsc.get_sparse_core_info()
        return info.num_cores, info.num_subcores
    except Exception:
        return 2, 16


@functools.lru_cache(maxsize=None)
def _build(BATCH, SEQ, V, D):
    NC, NS = _sc_geometry()
    NW = NC * NS
    assert BATCH % NW == 0
    rows_per_w = BATCH // NW
    NBUF = 4
    assert rows_per_w % NBUF == 0
    n_steps = rows_per_w // NBUF
    scale = math.sqrt(D)
    assert D % 16 == 0
    d_vecs = D // 16
    assert 128 < SEQ <= 256
    REM = SEQ - 128

    mesh = plsc.VectorSubcoreMesh(core_axis_name="c", subcore_axis_name="s")

    @functools.partial(
        pl.kernel,
        out_type=jax.ShapeDtypeStruct((BATCH * SEQ, D), jnp.float32),
        mesh=mesh,
        scratch_types=[
            pltpu.VMEM((rows_per_w, SEQ), jnp.int32),
            [pltpu.VMEM((SEQ, D), jnp.float32) for _ in range(NBUF)],
            [pltpu.VMEM((SEQ, D), jnp.float32) for _ in range(NBUF)],
            [pltpu.SemaphoreType.DMA for _ in range(NBUF)],
            [pltpu.SemaphoreType.DMA for _ in range(NBUF)],
        ],
        compiler_params=pltpu.CompilerParams(use_tc_tiling_on_sc=False),
    )
    def emb_kernel(tokens_hbm, table_hbm, out_hbm, idx_v, rows_g, rows_s,
                   sem_g, sem_s):
        wid = lax.axis_index("s") * NC + lax.axis_index("c")
        row0 = wid * rows_per_w

        pltpu.sync_copy(tokens_hbm.at[pl.ds(row0, rows_per_w)], idx_v)

        def start_gather(r, b):
            pltpu.async_copy(
                table_hbm.at[idx_v.at[r, pl.ds(0, 128)]],
                rows_g[b].at[pl.ds(0, 128)], sem_g[b])
            pltpu.async_copy(
                table_hbm.at[idx_v.at[r, pl.ds(128, REM)]],
                rows_g[b].at[pl.ds(128, REM)], sem_g[b])

        def wait_gather(b):
            pltpu.make_async_copy(
                table_hbm.at[idx_v.at[0, pl.ds(0, 128)]],
                rows_g[b].at[pl.ds(0, 128)], sem_g[b]).wait()
            pltpu.make_async_copy(
                table_hbm.at[idx_v.at[0, pl.ds(128, REM)]],
                rows_g[b].at[pl.ds(128, REM)], sem_g[b]).wait()

        def wait_scatter(b):
            pltpu.make_async_copy(
                rows_s[b], out_hbm.at[pl.ds(0, SEQ)], sem_s[b]).wait()

        for b in range(NBUF):
            start_gather(b, b)

        def step_body(step, _):
            for b in range(NBUF):
                r = step * NBUF + b
                wait_gather(b)

                @pl.when(step > 0)
                def _wait_prev_scatter(b=b):
                    wait_scatter(b)

                src, dst = rows_g[b], rows_s[b]

                @pl.loop(0, SEQ, unroll=8)
                def _scale(i):
                    for j in range(d_vecs):
                        sl = pl.ds(j * 16, 16)
                        dst[i, sl] = src[i, sl] * scale

                pltpu.async_copy(
                    dst, out_hbm.at[pl.ds((row0 + r) * SEQ, SEQ)], sem_s[b])

                r2 = r + NBUF

                @pl.when(r2 < rows_per_w)
                def _next_gather(r2=r2, b=b):
                    start_gather(r2, b)
            return ()

        lax.fori_loop(0, n_steps, step_body, ())

        for b in range(NBUF):
            wait_scatter(b)

    return emb_kernel


def kernel(tokens, table):
    batch, seq = tokens.shape
    V, D = table.shape
    out = _build(batch, seq, V, D)(tokens.astype(jnp.int32), table)
    return out.reshape(batch, seq, D)


# speedup vs baseline: 1.0006x; 1.0006x over previous
import functools
import math

import jax
import jax.numpy as jnp
from jax import lax
from jax.experimental import pallas as pl
from jax.experimental.pallas import tpu as pltpu
from jax.experimental.pallas import tpu_sc as plsc


def _sc_geometry():
    try:
        info = plsc.get_sparse_core_info()
        return info.num_cores, info.num_subcores
    except Exception:
        return 2, 16


@functools.lru_cache(maxsize=None)
def _build(BATCH, SEQ, V, D):
    NC, NS = _sc_geometry()
    NW = NC * NS
    assert BATCH % NW == 0
    rows_per_w = BATCH // NW
    NBUF = 4
    assert rows_per_w % NBUF == 0
    n_steps = rows_per_w // NBUF
    scale = math.sqrt(D)
    assert D % 16 == 0
    d_vecs = D // 16
    assert 128 < SEQ <= 256
    REM = SEQ - 128

    mesh = plsc.VectorSubcoreMesh(core_axis_name="c", subcore_axis_name="s")

    @functools.partial(
        pl.kernel,
        out_type=jax.ShapeDtypeStruct((BATCH, SEQ, D), jnp.float32),
        mesh=mesh,
        scratch_types=[
            pltpu.VMEM((rows_per_w, SEQ), jnp.int32),
            [pltpu.VMEM((SEQ, D), jnp.float32) for _ in range(NBUF)],
            [pltpu.VMEM((SEQ, D), jnp.float32) for _ in range(NBUF)],
            [pltpu.SemaphoreType.DMA for _ in range(NBUF)],
            [pltpu.SemaphoreType.DMA for _ in range(NBUF)],
        ],
        compiler_params=pltpu.CompilerParams(use_tc_tiling_on_sc=False),
    )
    def emb_kernel(tokens_hbm, table_hbm, out_hbm, idx_v, rows_g, rows_s,
                   sem_g, sem_s):
        wid = lax.axis_index("s") * NC + lax.axis_index("c")
        row0 = wid * rows_per_w

        pltpu.sync_copy(tokens_hbm.at[pl.ds(row0, rows_per_w)], idx_v)

        def start_gather(r, b):
            pltpu.async_copy(
                table_hbm.at[idx_v.at[r, pl.ds(0, 128)]],
                rows_g[b].at[pl.ds(0, 128)], sem_g[b])
            pltpu.async_copy(
                table_hbm.at[idx_v.at[r, pl.ds(128, REM)]],
                rows_g[b].at[pl.ds(128, REM)], sem_g[b])

        def wait_gather(b):
            pltpu.make_async_copy(
                table_hbm.at[idx_v.at[0, pl.ds(0, 128)]],
                rows_g[b].at[pl.ds(0, 128)], sem_g[b]).wait()
            pltpu.make_async_copy(
                table_hbm.at[idx_v.at[0, pl.ds(128, REM)]],
                rows_g[b].at[pl.ds(128, REM)], sem_g[b]).wait()

        def wait_scatter(b):
            pltpu.make_async_copy(
                rows_s[b], out_hbm.at[0], sem_s[b]).wait()

        for b in range(NBUF):
            start_gather(b, b)

        def step_body(step, _):
            for b in range(NBUF):
                r = step * NBUF + b
                wait_gather(b)

                @pl.when(step > 0)
                def _wait_prev_scatter(b=b):
                    wait_scatter(b)

                src, dst = rows_g[b], rows_s[b]

                @pl.loop(0, SEQ, unroll=8)
                def _scale(i):
                    for j in range(d_vecs):
                        sl = pl.ds(j * 16, 16)
                        dst[i, sl] = src[i, sl] * scale

                pltpu.async_copy(dst, out_hbm.at[row0 + r], sem_s[b])

                r2 = r + NBUF

                @pl.when(r2 < rows_per_w)
                def _next_gather(r2=r2, b=b):
                    start_gather(r2, b)
            return ()

        lax.fori_loop(0, n_steps, step_body, ())

        for b in range(NBUF):
            wait_scatter(b)

    return emb_kernel


def kernel(tokens, table):
    batch, seq = tokens.shape
    V, D = table.shape
    return _build(batch, seq, V, D)(tokens.astype(jnp.int32), table)


# speedup vs baseline: 1.3059x; 1.3052x over previous
import functools
import math

import jax
import jax.numpy as jnp
from jax import lax
from jax.experimental import pallas as pl
from jax.experimental.pallas import tpu as pltpu
from jax.experimental.pallas import tpu_sc as plsc


def _sc_geometry():
    try:
        info = plsc.get_sparse_core_info()
        return info.num_cores, info.num_subcores
    except Exception:
        return 2, 16


@functools.lru_cache(maxsize=None)
def _build(BATCH, SEQ, V, D):
    NC, NS = _sc_geometry()
    NW = NC * NS
    assert BATCH % NW == 0
    rows_per_w = BATCH // NW
    NBUF = 4
    scale = math.sqrt(D)
    assert D % 16 == 0
    d_vecs = D // 16
    assert 128 < SEQ <= 256
    REM = SEQ - 128
    assert rows_per_w % NBUF == 0
    n_steps = rows_per_w // NBUF

    mesh = plsc.VectorSubcoreMesh(core_axis_name="c", subcore_axis_name="s")

    @functools.partial(
        pl.kernel,
        out_type=jax.ShapeDtypeStruct((BATCH * SEQ, 2 * D), jnp.float32),
        mesh=mesh,
        scratch_types=[
            pltpu.VMEM((rows_per_w, SEQ), jnp.int32),
            [pltpu.VMEM((SEQ,), jnp.int32) for _ in range(NBUF)],
            [pltpu.VMEM((SEQ, D), jnp.float32) for _ in range(NBUF)],
            [pltpu.VMEM((SEQ, D), jnp.float32) for _ in range(NBUF)],
            [pltpu.SemaphoreType.DMA for _ in range(NBUF)],
            [pltpu.SemaphoreType.DMA for _ in range(NBUF)],
        ],
        compiler_params=pltpu.CompilerParams(use_tc_tiling_on_sc=False),
    )
    def emb_kernel(tokens_hbm, table_hbm, out_hbm, idx_v, idx2, rows_g,
                   rows_s, sem_g, sem_s):
        wid = lax.axis_index("s") * NC + lax.axis_index("c")
        row0 = wid * rows_per_w

        pltpu.sync_copy(tokens_hbm.at[pl.ds(row0, rows_per_w)], idx_v)

        def start_gather(r, b):
            starts = [16 * k for k in range(SEQ // 16)] + [SEQ - 16]
            for c in starts:
                sl = pl.ds(c, 16)
                idx2[b][sl] = idx_v[r, sl] * 2
            pltpu.async_copy(
                table_hbm.at[idx2[b].at[pl.ds(0, 128)]],
                rows_g[b].at[pl.ds(0, 128)], sem_g[b])
            pltpu.async_copy(
                table_hbm.at[idx2[b].at[pl.ds(128, REM)]],
                rows_g[b].at[pl.ds(128, REM)], sem_g[b])

        def wait_gather(b):
            pltpu.make_async_copy(
                table_hbm.at[idx2[b].at[pl.ds(0, 128)]],
                rows_g[b].at[pl.ds(0, 128)], sem_g[b]).wait()
            pltpu.make_async_copy(
                table_hbm.at[idx2[b].at[pl.ds(128, REM)]],
                rows_g[b].at[pl.ds(128, REM)], sem_g[b]).wait()

        def start_scatter(r, b):
            pltpu.async_copy(
                rows_s[b],
                out_hbm.at[pl.ds((row0 + r) * SEQ, SEQ), pl.ds(0, D)],
                sem_s[b])

        def wait_scatter(b):
            pltpu.make_async_copy(
                rows_s[b], out_hbm.at[pl.ds(0, SEQ), pl.ds(0, D)],
                sem_s[b]).wait()

        for b in range(NBUF):
            start_gather(b, b)

        def step_body(step, _):
            for b in range(NBUF):
                r = step * NBUF + b
                wait_gather(b)

                @pl.when(step > 0)
                def _wait_prev_scatter(b=b):
                    wait_scatter(b)

                src, dst = rows_g[b], rows_s[b]

                @pl.loop(0, SEQ, unroll=8)
                def _scale(i):
                    for j in range(d_vecs):
                        sl = pl.ds(j * 16, 16)
                        dst[i, sl] = src[i, sl] * scale

                start_scatter(r, b)

                r2 = r + NBUF

                @pl.when(r2 < rows_per_w)
                def _next_gather(r2=r2, b=b):
                    start_gather(r2, b)
            return ()

        lax.fori_loop(0, n_steps, step_body, ())

        for b in range(NBUF):
            wait_scatter(b)

    return emb_kernel


def kernel(tokens, table):
    batch, seq = tokens.shape
    V, D = table.shape
    table2 = jnp.pad(table, ((0, 0), (0, D))).reshape(2 * V, D)
    out = _build(batch, seq, V, D)(tokens.astype(jnp.int32), table2)
    return out.reshape(batch, seq, 2 * D)[:, :, :D]
